# Initial kernel scaffold; baseline (speedup 1.0000x reference)
import functools

import jax
import jax.numpy as jnp
from jax.experimental import pallas as pl

N_NODES = 256
HEADS = 4
HID = 64
OUT = 64


def _gat_kernel(x_ref, w_ref, asrc_ref, adst_ref, bias_ref, fcw_ref, fcb_ref,
                g_ref, b_ref, m_ref, v_ref, out_ref):
    h = jax.lax.dot_general(
        x_ref[...], w_ref[...],
        dimension_numbers=(((0,), (0,)), ((), ())),
        preferred_element_type=jnp.float32)

    a_srcT = jax.lax.dot_general(
        asrc_ref[...], h,
        dimension_numbers=(((0,), (1,)), ((), ())),
        preferred_element_type=jnp.float32)
    a_dst = jax.lax.dot_general(
        h, adst_ref[...],
        dimension_numbers=(((1,), (0,)), ((), ())),
        preferred_element_type=jnp.float32)

    parts = []
    for hd in range(HEADS):
        s = a_dst[:, hd:hd + 1] + a_srcT[hd:hd + 1, :]
        s = jnp.where(s >= 0.0, s, 0.2 * s)
        smax = jnp.max(s, axis=1, keepdims=True)
        e = jnp.exp(s - smax)
        denom = jnp.sum(e, axis=1, keepdims=True)
        p = e * (1.0 / (denom + 1e-16))
        agg = jax.lax.dot_general(
            p, h[:, hd * HID:(hd + 1) * HID],
            dimension_numbers=(((1,), (0,)), ((), ())),
            preferred_element_type=jnp.float32)
        parts.append(agg)
    agg_full = jnp.concatenate(parts, axis=1)

    r = jnp.maximum(agg_full + bias_ref[...], 0.0)
    mean_r = jnp.mean(r, axis=0, keepdims=True)

    y = jax.lax.dot_general(
        mean_r, fcw_ref[...],
        dimension_numbers=(((1,), (1,)), ((), ())),
        preferred_element_type=jnp.float32) + fcb_ref[...]
    scale = g_ref[...] * jax.lax.rsqrt(v_ref[...] + 1e-5)
    out_ref[...] = (y - m_ref[...]) * scale + b_ref[...]


@functools.partial(jax.jit, static_argnames=())
def kernel(x, W_gat, att_src, att_dst, bias_gat, fc_W, fc_b,
           bn_gamma, bn_beta, bn_mean, bn_var):
    eye = jnp.eye(HEADS, dtype=jnp.float32)
    A_src = (att_src.reshape(HEADS, HID)[:, :, None]
             * eye[:, None, :]).reshape(HEADS * HID, HEADS)
    A_dst = (att_dst.reshape(HEADS, HID)[:, :, None]
             * eye[:, None, :]).reshape(HEADS * HID, HEADS)

    out = pl.pallas_call(
        _gat_kernel,
        out_shape=jax.ShapeDtypeStruct((1, OUT), jnp.float32),
    )(x, W_gat, A_src, A_dst,
      bias_gat.reshape(1, HEADS * HID), fc_W, fc_b.reshape(1, OUT),
      bn_gamma.reshape(1, OUT), bn_beta.reshape(1, OUT),
      bn_mean.reshape(1, OUT), bn_var.reshape(1, OUT))
    return out.reshape(OUT)


# speedup vs baseline: 1334.0028x; 1334.0028x over previous
import functools

import jax
import jax.numpy as jnp
from jax.experimental import pallas as pl

N_NODES = 256
HEADS = 4
HID = 64
OUT = 64


def _gat_kernel(x_ref, w_ref, asrc_ref, adst_ref, bias_ref, fcw_ref, fcb_ref,
                g_ref, b_ref, m_ref, v_ref, out_ref):
    h = jax.lax.dot_general(
        x_ref[...], w_ref[...],
        dimension_numbers=(((0,), (0,)), ((), ())),
        preferred_element_type=jnp.float32)

    a_srcT = jax.lax.dot_general(
        asrc_ref[...], h,
        dimension_numbers=(((0,), (1,)), ((), ())),
        preferred_element_type=jnp.float32)
    a_dst = jax.lax.dot_general(
        h, adst_ref[...],
        dimension_numbers=(((1,), (0,)), ((), ())),
        preferred_element_type=jnp.float32)

    ones_col = jnp.ones((N_NODES, 1), dtype=jnp.float32)
    parts = []
    for hd in range(HEADS):
        s = a_dst[:, hd:hd + 1] + a_srcT[hd:hd + 1, :]
        s = jnp.where(s >= 0.0, s, 0.2 * s)
        smax = jnp.max(s, axis=1, keepdims=True)
        e = jnp.exp(s - smax)
        hp = jnp.concatenate(
            [h[:, hd * HID:(hd + 1) * HID], ones_col], axis=1)
        un = jax.lax.dot_general(
            e, hp,
            dimension_numbers=(((1,), (0,)), ((), ())),
            preferred_element_type=jnp.float32)
        agg = un[:, :HID] * (1.0 / (un[:, HID:HID + 1] + 1e-16))
        parts.append(agg)
    agg_full = jnp.concatenate(parts, axis=1)

    r = jnp.maximum(agg_full + bias_ref[...], 0.0)
    mean_r = jnp.mean(r, axis=0, keepdims=True)

    y = jax.lax.dot_general(
        mean_r, fcw_ref[...],
        dimension_numbers=(((1,), (1,)), ((), ())),
        preferred_element_type=jnp.float32) + fcb_ref[...]
    scale = g_ref[...] * jax.lax.rsqrt(v_ref[...] + 1e-5)
    out_ref[...] = (y - m_ref[...]) * scale + b_ref[...]


@functools.partial(jax.jit, static_argnames=())
def kernel(x, W_gat, att_src, att_dst, bias_gat, fc_W, fc_b,
           bn_gamma, bn_beta, bn_mean, bn_var):
    eye = jnp.eye(HEADS, dtype=jnp.float32)
    A_src = (att_src.reshape(HEADS, HID)[:, :, None]
             * eye[:, None, :]).reshape(HEADS * HID, HEADS)
    A_dst = (att_dst.reshape(HEADS, HID)[:, :, None]
             * eye[:, None, :]).reshape(HEADS * HID, HEADS)

    out = pl.pallas_call(
        _gat_kernel,
        out_shape=jax.ShapeDtypeStruct((1, OUT), jnp.float32),
    )(x, W_gat, A_src, A_dst,
      bias_gat.reshape(1, HEADS * HID), fc_W, fc_b.reshape(1, OUT),
      bn_gamma.reshape(1, OUT), bn_beta.reshape(1, OUT),
      bn_mean.reshape(1, OUT), bn_var.reshape(1, OUT))
    return out.reshape(OUT)


# speedup vs baseline: 1777.7698x; 1.3327x over previous
import functools

import jax
import jax.numpy as jnp
from jax.experimental import pallas as pl

N_NODES = 256
HEADS = 4
HID = 64
OUT = 64


def _gat_kernel(x_ref, w_ref, asrc_ref, adst_ref, bias_ref, fcw_ref, fcb_ref,
                g_ref, b_ref, m_ref, v_ref, out_ref):
    h = jax.lax.dot_general(
        x_ref[...], w_ref[...],
        dimension_numbers=(((0,), (0,)), ((), ())),
        preferred_element_type=jnp.float32)

    ones_col = jnp.ones((N_NODES, 1), dtype=jnp.float32)
    parts = []
    for hd in range(HEADS):
        h_hd = h[:, hd * HID:(hd + 1) * HID]
        a_srcT = jax.lax.dot_general(
            asrc_ref[hd:hd + 1, :], h_hd,
            dimension_numbers=(((1,), (1,)), ((), ())),
            preferred_element_type=jnp.float32)
        a_dst = jax.lax.dot_general(
            h_hd, adst_ref[hd:hd + 1, :],
            dimension_numbers=(((1,), (1,)), ((), ())),
            preferred_element_type=jnp.float32)
        s = a_dst + a_srcT
        s = jnp.where(s >= 0.0, s, 0.2 * s)
        smax = jnp.max(s, axis=1, keepdims=True)
        e = jnp.exp(s - smax)
        hp = jnp.concatenate([h_hd, ones_col], axis=1)
        un = jax.lax.dot_general(
            e, hp,
            dimension_numbers=(((1,), (0,)), ((), ())),
            preferred_element_type=jnp.float32)
        agg = un[:, :HID] * (1.0 / (un[:, HID:HID + 1] + 1e-16))
        parts.append(agg)
    agg_full = jnp.concatenate(parts, axis=1)

    r = jnp.maximum(agg_full + bias_ref[...], 0.0)
    mean_r = jnp.mean(r, axis=0, keepdims=True)

    y = jax.lax.dot_general(
        mean_r, fcw_ref[...],
        dimension_numbers=(((1,), (1,)), ((), ())),
        preferred_element_type=jnp.float32) + fcb_ref[...]
    scale = g_ref[...] * jax.lax.rsqrt(v_ref[...] + 1e-5)
    out_ref[...] = (y - m_ref[...]) * scale + b_ref[...]


@functools.partial(jax.jit, static_argnames=())
def kernel(x, W_gat, att_src, att_dst, bias_gat, fc_W, fc_b,
           bn_gamma, bn_beta, bn_mean, bn_var):
    out = pl.pallas_call(
        _gat_kernel,
        out_shape=jax.ShapeDtypeStruct((1, OUT), jnp.float32),
    )(x, W_gat, att_src.reshape(HEADS, HID), att_dst.reshape(HEADS, HID),
      bias_gat.reshape(1, HEADS * HID), fc_W, fc_b.reshape(1, OUT),
      bn_gamma.reshape(1, OUT), bn_beta.reshape(1, OUT),
      bn_mean.reshape(1, OUT), bn_var.reshape(1, OUT))
    return out.reshape(OUT)
